# baseline (speedup 1.0000x reference)
import jax
import jax.numpy as jnp
from jax.experimental import pallas as pl

QB = 256
M = 16384
NCHUNK = 128
CW = 128
K = 3


def _top3(vals, tie, big_tie, inf):
    m1 = jnp.min(vals, axis=1, keepdims=True)
    t1 = jnp.min(jnp.where(vals == m1, tie, big_tie), axis=1, keepdims=True)
    e1 = tie == t1
    m2 = jnp.min(jnp.where(e1, inf, vals), axis=1, keepdims=True)
    t2 = jnp.min(jnp.where((vals == m2) & ~e1, tie, big_tie),
                 axis=1, keepdims=True)
    e2 = e1 | (tie == t2)
    m3 = jnp.min(jnp.where(e2, inf, vals), axis=1, keepdims=True)
    t3 = jnp.min(jnp.where((vals == m3) & ~e2, tie, big_tie),
                 axis=1, keepdims=True)
    return m1, t1, m2, t2, m3, t3


def _body(q_ref, packed_ref, rt2_ref, out_ref):
    q = q_ref[...]
    packed = packed_ref[...]
    rt2 = rt2_ref[...]

    inf = jnp.float32(jnp.inf)
    big = jnp.int32(2**30)

    q2 = jnp.sum(q * q, axis=1, keepdims=True)
    r2 = packed[3:4, :]
    qr = jnp.dot(q, packed, preferred_element_type=jnp.float32)
    d2 = q2 - 2.0 * qr + r2

    cm = jnp.min(d2.reshape(QB, NCHUNK, CW), axis=2)

    ci = jax.lax.broadcasted_iota(jnp.int32, (QB, NCHUNK), 1)
    _, c1, _, c2, _, c3 = _top3(cm, ci, big, inf)

    gs = []
    for ck in (c1, c2, c3):
        oh = (ci == ck).astype(jnp.float32)
        gs.append(jnp.dot(oh, rt2, preferred_element_type=jnp.float32))

    qx, qy, qz = q[:, 0:1], q[:, 1:2], q[:, 2:3]
    dists, gidxs, flows = [], [], []
    liota = jax.lax.broadcasted_iota(jnp.int32, (QB, CW), 1)
    for ck, g in zip((c1, c2, c3), gs):
        rx = g[:, 0 * CW:1 * CW]
        ry = g[:, 1 * CW:2 * CW]
        rz = g[:, 2 * CW:3 * CW]
        rr2 = g[:, 3 * CW:4 * CW]
        d2c = q2 - 2.0 * (qx * rx + qy * ry + qz * rz) + rr2
        dists.append(jnp.sqrt(jnp.maximum(d2c, 0.0)))
        gidxs.append(ck * CW + liota)
        flows.append(g[:, 4 * CW:7 * CW])

    dcand = jnp.concatenate(dists, axis=1)
    gcand = jnp.concatenate(gidxs, axis=1)

    m1, g1, m2, g2, m3, g3 = _top3(dcand, gcand, big, inf)

    w1 = 1.0 / (m1 + 1e-8)
    w2 = 1.0 / (m2 + 1e-8)
    w3 = 1.0 / (m3 + 1e-8)
    wsum = w1 + w2 + w3
    zero = jnp.float32(0.0)
    wrow = (jnp.where(gcand == g1, w1, zero)
            + jnp.where(gcand == g2, w2, zero)
            + jnp.where(gcand == g3, w3, zero))

    for c in range(3):
        fc = jnp.concatenate(
            [f[:, c * CW:(c + 1) * CW] for f in flows], axis=1)
        out_ref[:, c:c + 1] = (
            jnp.sum(wrow * fc, axis=1, keepdims=True) / wsum)
    out_ref[:, 3:] = jnp.zeros((QB, 5), jnp.float32)


@jax.jit
def kernel(query_points, ref_points, ref_flow):
    n = query_points.shape[0]
    qp = jnp.zeros((n, 8), jnp.float32).at[:, :3].set(query_points)
    r2 = jnp.sum(ref_points * ref_points, axis=1)
    packed = jnp.concatenate(
        [ref_points.T, r2[None, :], ref_flow.T,
         jnp.zeros((1, M), jnp.float32)], axis=0)
    rt2 = packed.reshape(8, NCHUNK, CW).transpose(1, 0, 2).reshape(NCHUNK, 8 * CW)

    out = pl.pallas_call(
        _body,
        grid=(n // QB,),
        in_specs=[
            pl.BlockSpec((QB, 8), lambda i: (i, 0)),
            pl.BlockSpec((8, M), lambda i: (0, 0)),
            pl.BlockSpec((NCHUNK, 8 * CW), lambda i: (0, 0)),
        ],
        out_specs=pl.BlockSpec((QB, 8), lambda i: (i, 0)),
        out_shape=jax.ShapeDtypeStruct((n, 8), jnp.float32),
    )(qp, packed, rt2)
    return out[:, :3]


# speedup vs baseline: 6.6817x; 6.6817x over previous
import jax
import jax.numpy as jnp
from jax.experimental import pallas as pl

QB = 256
M = 16384
NCHUNK = 128
CW = 128
K = 3


def _top3(vals, tie, big_tie, inf):
    m1 = jnp.min(vals, axis=1, keepdims=True)
    t1 = jnp.min(jnp.where(vals == m1, tie, big_tie), axis=1, keepdims=True)
    e1 = tie == t1
    m2 = jnp.min(jnp.where(e1, inf, vals), axis=1, keepdims=True)
    t2 = jnp.min(jnp.where((vals == m2) & ~e1, tie, big_tie),
                 axis=1, keepdims=True)
    e2 = e1 | (tie == t2)
    m3 = jnp.min(jnp.where(e2, inf, vals), axis=1, keepdims=True)
    t3 = jnp.min(jnp.where((vals == m3) & ~e2, tie, big_tie),
                 axis=1, keepdims=True)
    return m1, t1, m2, t2, m3, t3


def _body(q_ref, packed_ref, rt2_ref, out_ref):
    q = q_ref[...]
    packed = packed_ref[...]
    rt2 = rt2_ref[...]

    inf = jnp.float32(jnp.inf)
    big = jnp.int32(2**30)

    q2 = jnp.sum(q * q, axis=1, keepdims=True)
    r2 = packed[3:4, :]
    qr = jnp.dot(q, packed, preferred_element_type=jnp.float32)
    d2 = q2 - 2.0 * qr + r2

    cm = d2
    half = M // 2
    while half >= NCHUNK:
        cm = jnp.minimum(cm[:, :half], cm[:, half:2 * half])
        half //= 2

    ci = jax.lax.broadcasted_iota(jnp.int32, (QB, NCHUNK), 1)
    _, c1, _, c2, _, c3 = _top3(cm, ci, big, inf)

    gs = []
    for ck in (c1, c2, c3):
        oh = (ci == ck).astype(jnp.float32)
        gs.append(jnp.dot(oh, rt2, preferred_element_type=jnp.float32))

    qx, qy, qz = q[:, 0:1], q[:, 1:2], q[:, 2:3]
    dists, gidxs, flows = [], [], []
    liota = jax.lax.broadcasted_iota(jnp.int32, (QB, CW), 1)
    for ck, g in zip((c1, c2, c3), gs):
        rx = g[:, 0 * CW:1 * CW]
        ry = g[:, 1 * CW:2 * CW]
        rz = g[:, 2 * CW:3 * CW]
        rr2 = g[:, 3 * CW:4 * CW]
        d2c = q2 - 2.0 * (qx * rx + qy * ry + qz * rz) + rr2
        dists.append(jnp.sqrt(jnp.maximum(d2c, 0.0)))
        gidxs.append(liota * NCHUNK + ck)
        flows.append(g[:, 4 * CW:7 * CW])

    dcand = jnp.concatenate(dists, axis=1)
    gcand = jnp.concatenate(gidxs, axis=1)

    m1, g1, m2, g2, m3, g3 = _top3(dcand, gcand, big, inf)

    w1 = 1.0 / (m1 + 1e-8)
    w2 = 1.0 / (m2 + 1e-8)
    w3 = 1.0 / (m3 + 1e-8)
    wsum = w1 + w2 + w3
    zero = jnp.float32(0.0)
    wrow = (jnp.where(gcand == g1, w1, zero)
            + jnp.where(gcand == g2, w2, zero)
            + jnp.where(gcand == g3, w3, zero))

    for c in range(3):
        fc = jnp.concatenate(
            [f[:, c * CW:(c + 1) * CW] for f in flows], axis=1)
        out_ref[:, c:c + 1] = (
            jnp.sum(wrow * fc, axis=1, keepdims=True) / wsum)
    out_ref[:, 3:] = jnp.zeros((QB, 5), jnp.float32)


@jax.jit
def kernel(query_points, ref_points, ref_flow):
    n = query_points.shape[0]
    qp = jnp.zeros((n, 8), jnp.float32).at[:, :3].set(query_points)
    r2 = jnp.sum(ref_points * ref_points, axis=1)
    packed = jnp.concatenate(
        [ref_points.T, r2[None, :], ref_flow.T,
         jnp.zeros((1, M), jnp.float32)], axis=0)
    rt2 = packed.reshape(8, CW, NCHUNK).transpose(2, 0, 1).reshape(NCHUNK, 8 * CW)

    out = pl.pallas_call(
        _body,
        grid=(n // QB,),
        in_specs=[
            pl.BlockSpec((QB, 8), lambda i: (i, 0)),
            pl.BlockSpec((8, M), lambda i: (0, 0)),
            pl.BlockSpec((NCHUNK, 8 * CW), lambda i: (0, 0)),
        ],
        out_specs=pl.BlockSpec((QB, 8), lambda i: (i, 0)),
        out_shape=jax.ShapeDtypeStruct((n, 8), jnp.float32),
    )(qp, packed, rt2)
    return out[:, :3]
